# Initial kernel scaffold; baseline (speedup 1.0000x reference)
import jax
import jax.numpy as jnp
from jax.experimental import pallas as pl


def kernel(macro_features, net_features, port_features, edge_index_m2n, edge_attr_m2n, edge_index_n2m, edge_attr_n2m, movable_mask, canvas_w, canvas_h, macro_sizes, params):
    raise NotImplementedError("write your pallas kernel here")


# speedup vs baseline: 1.4428x; 1.4428x over previous
import jax
import jax.numpy as jnp
from jax import lax
from jax.experimental import pallas as pl
from jax.experimental.pallas import tpu as pltpu
from jax.experimental.pallas import tpu_sc as plsc

DD = 64
NEDGE = 160000
NMAC = 9000
NPORTS = 1000
NNETS = 10000
NN = NMAC + NPORTS

NCORES = 2
NSUB = 16
NWORK = NCORES * NSUB
CHUNK = 128
NCHUNKS = NEDGE // CHUNK
TRIPS = -(-NCHUNKS // NWORK)
ROWS_PER_SUB = NNETS // NSUB

_MESH = plsc.VectorSubcoreMesh(core_axis_name="c", subcore_axis_name="s",
                               num_cores=NCORES, num_subcores=NSUB)
_SC_PARAMS = pltpu.CompilerParams(use_tc_tiling_on_sc=False)


def _sc_gather_body(a_hbm, b_hbm, src_hbm, dst_hbm, ga_hbm, gb_hbm,
                    idxa, idxb, bufa, bufb, sema, semb):
    c = lax.axis_index("c")
    s = lax.axis_index("s")
    w = c * NSUB + s

    def step(j, carry):
        ch = w + NWORK * j

        @pl.when(ch < NCHUNKS)
        def _():
            base = ch * CHUNK
            pltpu.sync_copy(src_hbm.at[pl.ds(base, CHUNK)], idxa)
            pltpu.sync_copy(dst_hbm.at[pl.ds(base, CHUNK)], idxb)
            da = pltpu.async_copy(a_hbm.at[idxa], bufa, sema)
            db = pltpu.async_copy(b_hbm.at[idxb], bufb, semb)
            da.wait()
            db.wait()
            pltpu.sync_copy(bufa, ga_hbm.at[pl.ds(base, CHUNK)])
            pltpu.sync_copy(bufb, gb_hbm.at[pl.ds(base, CHUNK)])

        return carry

    lax.fori_loop(0, TRIPS, step, 0)


_sc_gather = pl.kernel(
    _sc_gather_body,
    out_type=(jax.ShapeDtypeStruct((NEDGE, DD), jnp.float32),
              jax.ShapeDtypeStruct((NEDGE, DD), jnp.float32)),
    mesh=_MESH,
    scratch_types=[pltpu.VMEM((CHUNK,), jnp.int32),
                   pltpu.VMEM((CHUNK,), jnp.int32),
                   pltpu.VMEM((CHUNK, DD), jnp.float32),
                   pltpu.VMEM((CHUNK, DD), jnp.float32),
                   pltpu.SemaphoreType.DMA,
                   pltpu.SemaphoreType.DMA],
    compiler_params=_SC_PARAMS,
)


def _sc_scatter_body(y_hbm, dst_hbm, zero_hbm, out_hbm, idxb, ybuf, acc):
    c = lax.axis_index("c")
    s = lax.axis_index("s")
    w = c * NSUB + s

    pltpu.sync_copy(zero_hbm, acc.at[pl.ds(s * ROWS_PER_SUB, ROWS_PER_SUB)])
    plsc.subcore_barrier()

    def step(j, carry):
        ch = w + NWORK * j

        @pl.when(ch < NCHUNKS)
        def _():
            base = ch * CHUNK
            pltpu.sync_copy(dst_hbm.at[pl.ds(base, CHUNK)], idxb)
            pltpu.sync_copy(y_hbm.at[pl.ds(base, CHUNK)], ybuf)
            pltpu.sync_copy(ybuf, acc.at[idxb], add=True)

        return carry

    lax.fori_loop(0, TRIPS, step, 0)
    plsc.subcore_barrier()
    pltpu.sync_copy(acc.at[pl.ds(s * ROWS_PER_SUB, ROWS_PER_SUB)],
                    out_hbm.at[c, pl.ds(s * ROWS_PER_SUB, ROWS_PER_SUB)])


_sc_scatter = pl.kernel(
    _sc_scatter_body,
    out_type=jax.ShapeDtypeStruct((NCORES, NNETS, DD), jnp.float32),
    mesh=_MESH,
    scratch_types=[pltpu.VMEM((CHUNK,), jnp.int32),
                   pltpu.VMEM((CHUNK, DD), jnp.float32),
                   pltpu.VMEM_SHARED((NNETS, DD), jnp.float32)],
    compiler_params=_SC_PARAMS,
)


def _layernorm(x, g, b):
    m = x.mean(-1, keepdims=True)
    v = ((x - m) ** 2).mean(-1, keepdims=True)
    return (x - m) * lax.rsqrt(v + 1e-5) * g + b


def _project_body(mf, nf, pf, pmw, pmb, pnw, pnb, ppw, ppb, wsrc, wdst,
                  hn_out, hnet_out, a_out, b_out):
    hm = jnp.dot(mf[...], pmw[...], preferred_element_type=jnp.float32) + pmb[...]
    hp = jnp.dot(pf[...], ppw[...], preferred_element_type=jnp.float32) + ppb[...]
    hnet = jnp.dot(nf[...], pnw[...], preferred_element_type=jnp.float32) + pnb[...]
    hn = jnp.concatenate([hm, hp], axis=0)
    hn_out[...] = hn
    hnet_out[...] = hnet
    a_out[...] = jnp.dot(hn, wsrc[...], preferred_element_type=jnp.float32)
    b_out[...] = jnp.dot(hnet, wdst[...], preferred_element_type=jnp.float32)


def _tc_project(mf, nf, pf, pmw, pmb, pnw, pnb, ppw, ppb, wsrc, wdst):
    return pl.pallas_call(
        _project_body,
        out_shape=(jax.ShapeDtypeStruct((NN, DD), jnp.float32),
                   jax.ShapeDtypeStruct((NNETS, DD), jnp.float32),
                   jax.ShapeDtypeStruct((NN, DD), jnp.float32),
                   jax.ShapeDtypeStruct((NNETS, DD), jnp.float32)),
    )(mf, nf, pf, pmw, pmb, pnw, pnb, ppw, ppb, wsrc, wdst)


_EB = 1000


def _edge_body(ga, gb, ea, wet, b1, g, b, y_out):
    x = ga[...] + gb[...] + jnp.dot(ea[...], wet[...],
                                    preferred_element_type=jnp.float32) + b1[...]
    y = _layernorm(x, g[...], b[...])
    y_out[...] = jnp.maximum(y, 0.0)


def _tc_edge(ga, gb, ea, wet, b1, g, b):
    blk = lambda shape: pl.BlockSpec(shape, lambda i: (0,) * len(shape))
    return pl.pallas_call(
        _edge_body,
        grid=(NEDGE // _EB,),
        in_specs=[pl.BlockSpec((_EB, DD), lambda i: (i, 0)),
                  pl.BlockSpec((_EB, DD), lambda i: (i, 0)),
                  pl.BlockSpec((_EB, 3), lambda i: (i, 0)),
                  blk((3, DD)), blk((1, DD)), blk((1, DD)), blk((1, DD))],
        out_specs=pl.BlockSpec((_EB, DD), lambda i: (i, 0)),
        out_shape=jax.ShapeDtypeStruct((NEDGE, DD), jnp.float32),
    )(ga, gb, ea, wet, b1, g, b)


_NB = 1000


def _finalize_body(part, pcnt, h, other, w2t, b2, wiht, bih, whht, bhh,
                   ng, nb, wsnt, wdnt, h_out, a_out, b_out):
    p = part[...]
    psum = p[0] + p[1]
    pc = pcnt[...]
    cnt = pc[0, :, 0:1] + pc[1, :, 0:1]
    mean = psum / jnp.maximum(cnt, 1.0)
    agg = jnp.dot(mean, w2t[...], preferred_element_type=jnp.float32) + b2[...]
    agg = jnp.where(cnt > 0.0, agg, 0.0)
    hv = h[...]
    gi = jnp.dot(agg, wiht[...], preferred_element_type=jnp.float32) + bih[...]
    gh = jnp.dot(hv, whht[...], preferred_element_type=jnp.float32) + bhh[...]
    r = jax.nn.sigmoid(gi[:, 0:DD] + gh[:, 0:DD])
    z = jax.nn.sigmoid(gi[:, DD:2 * DD] + gh[:, DD:2 * DD])
    n = jnp.tanh(gi[:, 2 * DD:3 * DD] + r * gh[:, 2 * DD:3 * DD])
    hn = _layernorm((1.0 - z) * n + z * hv, ng[...], nb[...])
    h_out[...] = hn
    a_out[...] = jnp.dot(hn, wsnt[...], preferred_element_type=jnp.float32)
    b_out[...] = jnp.dot(other[...], wdnt[...], preferred_element_type=jnp.float32)


def _tc_finalize(part, pcnt, h, other, w2t, b2, wiht, bih, whht, bhh,
                 ng, nb, wsnt, wdnt):
    nrows = h.shape[0]
    blk = lambda shape: pl.BlockSpec(shape, lambda i: (0,) * len(shape))
    return pl.pallas_call(
        _finalize_body,
        grid=(nrows // _NB,),
        in_specs=[pl.BlockSpec((NCORES, _NB, DD), lambda i: (0, i, 0)),
                  pl.BlockSpec((NCORES, _NB, DD), lambda i: (0, i, 0)),
                  pl.BlockSpec((_NB, DD), lambda i: (i, 0)),
                  pl.BlockSpec((_NB, DD), lambda i: (i, 0)),
                  blk((DD, DD)), blk((1, DD)),
                  blk((DD, 3 * DD)), blk((1, 3 * DD)),
                  blk((DD, 3 * DD)), blk((1, 3 * DD)),
                  blk((1, DD)), blk((1, DD)),
                  blk((DD, DD)), blk((DD, DD))],
        out_specs=(pl.BlockSpec((_NB, DD), lambda i: (i, 0)),
                   pl.BlockSpec((_NB, DD), lambda i: (i, 0)),
                   pl.BlockSpec((_NB, DD), lambda i: (i, 0))),
        out_shape=(jax.ShapeDtypeStruct((nrows, DD), jnp.float32),
                   jax.ShapeDtypeStruct((nrows, DD), jnp.float32),
                   jax.ShapeDtypeStruct((nrows, DD), jnp.float32)),
    )(part, pcnt, h, other, w2t, b2, wiht, bih, whht, bhh, ng, nb, wsnt, wdnt)


def _head_body(hm, w1t, b1, w2t, b2, scale, out):
    hidden = jnp.maximum(
        jnp.dot(hm[...], w1t[...], preferred_element_type=jnp.float32) + b1[...], 0.0)
    raw = jnp.dot(hidden, w2t[...], preferred_element_type=jnp.float32) + b2[...]
    sc = scale[...]
    out[...] = jax.nn.sigmoid(raw) * sc[:, 0:2] + sc[:, 2:4]


def _tc_head(hm, w1t, b1, w2t, b2, scale):
    blk = lambda shape: pl.BlockSpec(shape, lambda i: (0,) * len(shape))
    return pl.pallas_call(
        _head_body,
        grid=(NMAC // _NB,),
        in_specs=[pl.BlockSpec((_NB, DD), lambda i: (i, 0)),
                  blk((DD, DD)), blk((1, DD)), blk((DD, 2)), blk((1, 2)),
                  pl.BlockSpec((_NB, 4), lambda i: (i, 0))],
        out_specs=pl.BlockSpec((_NB, 2), lambda i: (i, 0)),
        out_shape=jax.ShapeDtypeStruct((NMAC, 2), jnp.float32),
    )(hm, w1t, b1, w2t, b2, scale)


def _prep_dir(lp, tag):
    w1 = lp[tag + "_l1"]["w"]
    return {
        "wsrc": w1[:, 0:DD].T,
        "wdst": w1[:, DD:2 * DD].T,
        "wet": w1[:, 2 * DD:].T,
        "b1": lp[tag + "_l1"]["b"].reshape(1, DD),
        "g": lp[tag + "_ln"]["g"].reshape(1, DD),
        "b": lp[tag + "_ln"]["b"].reshape(1, DD),
        "w2t": lp[tag + "_l2"]["w"].T,
        "b2": lp[tag + "_l2"]["b"].reshape(1, DD),
    }


def _prep_gru(gp):
    return (gp["w_ih"].T, gp["b_ih"].reshape(1, 3 * DD),
            gp["w_hh"].T, gp["b_hh"].reshape(1, 3 * DD))


def kernel(macro_features, net_features, port_features, edge_index_m2n,
           edge_attr_m2n, edge_index_n2m, edge_attr_n2m, movable_mask,
           canvas_w, canvas_h, macro_sizes, params):
    f32 = jnp.float32
    src_m2n = edge_index_m2n[0]
    dst_m2n = edge_index_m2n[1]
    src_n2m = edge_index_n2m[0]
    dst_n2m = edge_index_n2m[1]

    layers = params["layers"]
    m2n = [_prep_dir(lp, "m2n") for lp in layers]
    n2m = [_prep_dir(lp, "n2m") for lp in layers]

    zeros_blk = jnp.zeros((ROWS_PER_SUB, DD), f32)
    ones_e = jnp.ones((NEDGE, DD), f32)

    pcnt_net = _sc_scatter(ones_e, dst_m2n, zeros_blk)
    pcnt_mac = _sc_scatter(ones_e, dst_n2m, zeros_blk)

    h_nodes, h_net, a_tab, b_tab = _tc_project(
        macro_features, net_features, port_features,
        params["proj_macro"]["w"].T, params["proj_macro"]["b"].reshape(1, DD),
        params["proj_net"]["w"].T, params["proj_net"]["b"].reshape(1, DD),
        params["proj_port"]["w"].T, params["proj_port"]["b"].reshape(1, DD),
        m2n[0]["wsrc"], m2n[0]["wdst"])

    dummy = jnp.zeros((DD, DD), f32)
    for li in range(len(layers)):
        dm = m2n[li]
        dn = n2m[li]
        ga, gb = _sc_gather(a_tab, b_tab, src_m2n, dst_m2n)
        y = _tc_edge(ga, gb, edge_attr_m2n, dm["wet"], dm["b1"], dm["g"], dm["b"])
        part = _sc_scatter(y, dst_m2n, zeros_blk)
        wih, bih, whh, bhh = _prep_gru(layers[li]["gru_net"])
        h_net, a_tab, b_tab = _tc_finalize(
            part, pcnt_net, h_net, h_nodes, dm["w2t"], dm["b2"],
            wih, bih, whh, bhh,
            layers[li]["norm_net"]["g"].reshape(1, DD),
            layers[li]["norm_net"]["b"].reshape(1, DD),
            dn["wsrc"], dn["wdst"])
        ga, gb = _sc_gather(a_tab, b_tab, src_n2m, dst_n2m)
        y = _tc_edge(ga, gb, edge_attr_n2m, dn["wet"], dn["b1"], dn["g"], dn["b"])
        part = _sc_scatter(y, dst_n2m, zeros_blk)
        wih, bih, whh, bhh = _prep_gru(layers[li]["gru_macro"])
        nxt = m2n[li + 1] if li + 1 < len(layers) else None
        h_nodes, a_tab, b_tab = _tc_finalize(
            part, pcnt_mac, h_nodes, h_net, dn["w2t"], dn["b2"],
            wih, bih, whh, bhh,
            layers[li]["norm_macro"]["g"].reshape(1, DD),
            layers[li]["norm_macro"]["b"].reshape(1, DD),
            nxt["wsrc"] if nxt else dummy, nxt["wdst"] if nxt else dummy)

    cw = jnp.asarray(canvas_w, f32)
    ch = jnp.asarray(canvas_h, f32)
    half = macro_sizes * 0.5
    rng = jnp.clip(jnp.stack([cw - macro_sizes[:, 0], ch - macro_sizes[:, 1]],
                             axis=-1), 1.0, None)
    scale = jnp.concatenate([rng, half], axis=-1)
    return _tc_head(h_nodes[:NMAC], params["head_l1"]["w"].T,
                    params["head_l1"]["b"].reshape(1, DD),
                    params["head_l2"]["w"].T,
                    params["head_l2"]["b"].reshape(1, 2), scale)
